# baseline (speedup 1.0000x reference)
import functools

import jax
import jax.numpy as jnp
from jax import lax
from jax.experimental import pallas as pl
from jax.experimental.pallas import tpu as pltpu
from jax.experimental.pallas import tpu_sc as plsc

_N = 10000
_D = 128

_NC = 2
_NS = 16
_NW = _NC * _NS
_L = 16

_B = 128
_ROWS_ACC = 10112
_DEG_ACC = 10240
_RPH = 48
_BN = 2000


def _mesh():
    return plsc.VectorSubcoreMesh(
        core_axis_name="c", subcore_axis_name="s",
        num_cores=_NC, num_subcores=_NS)


def _ring(start, finish, cnt):
    m1 = (cnt - 1) // 2

    def body(i, carry):
        j = 2 * i
        start(j + 1, 1)
        finish(j, 0)
        start(j + 2, 0)
        finish(j + 1, 1)
        return carry

    if m1 > 0:
        lax.fori_loop(0, m1, body, 0)
    if cnt % 2 == 1:
        finish(cnt - 1, 0)
    else:
        if cnt >= 2:
            start(cnt - 1, 1)
            finish(cnt - 2, 0)
            finish(cnt - 1, 1)
        else:
            finish(0, 0)


@functools.lru_cache(maxsize=None)
def _make_sc_deg(nb):
    zlen = _DEG_ACC // _NS

    @functools.partial(
        pl.kernel,
        out_type=jax.ShapeDtypeStruct((_NC * _DEG_ACC,), jnp.float32),
        mesh=_mesh(),
        scratch_types=[
            pltpu.VMEM_SHARED((_DEG_ACC,), jnp.float32),
            pltpu.VMEM((nb, _B), jnp.int32),
            pltpu.VMEM((_B,), jnp.float32),
            pltpu.VMEM((zlen,), jnp.float32),
        ],
    )
    def deg_kernel(dst3, out, acc, idx_v, ones_v, zbuf_v):
        c = lax.axis_index("c")
        s = lax.axis_index("s")
        wid = c * _NS + s
        for k in range(_B // _L):
            ones_v[pl.ds(k * _L, _L)] = jnp.ones((_L,), jnp.float32)

        def zb(i, carry):
            zbuf_v[pl.ds(i * _L, _L)] = jnp.zeros((_L,), jnp.float32)
            return carry

        lax.fori_loop(0, zlen // _L, zb, 0)
        pltpu.sync_copy(zbuf_v, acc.at[pl.ds(s * zlen, zlen)])
        plsc.subcore_barrier()
        pltpu.sync_copy(dst3.at[wid], idx_v)

        def body(i, carry):
            pltpu.sync_copy(ones_v, acc.at[idx_v.at[i]], add=True)
            return carry

        lax.fori_loop(0, nb, body, 0)
        plsc.subcore_barrier()
        pltpu.sync_copy(acc.at[pl.ds(s * zlen, zlen)],
                        out.at[pl.ds(c * _DEG_ACC + s * zlen, zlen)])

    return deg_kernel


@functools.lru_cache(maxsize=None)
def _make_sc_agg(nb):
    zrows = _ROWS_ACC // _NS
    nph = -(-nb // _RPH)

    @functools.partial(
        pl.kernel,
        out_type=jax.ShapeDtypeStruct((_NC, _ROWS_ACC, _D), jnp.float32),
        mesh=_mesh(),
        scratch_types=[
            pltpu.VMEM_SHARED((_ROWS_ACC, _D), jnp.float32),
            pltpu.VMEM((_RPH, _B), jnp.int32),
            pltpu.VMEM((_RPH, _B), jnp.int32),
            pltpu.VMEM((_B, _D), jnp.float32),
            pltpu.VMEM((_B, _D), jnp.float32),
            pltpu.SemaphoreType.DMA,
        ],
    )
    def agg_kernel(y, src3, dst3, out, acc, sidx_v, didx_v, r0_v, r1_v, sem):
        c = lax.axis_index("c")
        s = lax.axis_index("s")
        wid = c * _NS + s

        def zb(i, carry):
            for k in range(_D // _L):
                r0_v[i, pl.ds(k * _L, _L)] = jnp.zeros((_L,), jnp.float32)
            return carry

        lax.fori_loop(0, _B, zb, 0)
        for j in range(zrows // _B):
            pltpu.sync_copy(r0_v, acc.at[pl.ds(s * zrows + j * _B, _B)])
        pltpu.sync_copy(r0_v.at[pl.ds(0, zrows % _B)],
                        acc.at[pl.ds(s * zrows + (zrows // _B) * _B,
                                     zrows % _B)])
        plsc.subcore_barrier()

        bufs = (r0_v, r1_v)
        for ph in range(nph):
            pb = ph * _RPH
            cnt = min(_RPH, nb - pb)
            pltpu.sync_copy(src3.at[wid, pl.ds(pb, cnt)],
                            sidx_v.at[pl.ds(0, cnt)])
            pltpu.sync_copy(dst3.at[wid, pl.ds(pb, cnt)],
                            didx_v.at[pl.ds(0, cnt)])

            def start(j, buf):
                pltpu.async_copy(y.at[sidx_v.at[j]], bufs[buf], sem)

            def finish(j, buf):
                pltpu.make_async_copy(y.at[sidx_v.at[j]], bufs[buf],
                                      sem).wait()
                pltpu.sync_copy(bufs[buf], acc.at[didx_v.at[j]], add=True)

            start(0, 0)
            _ring(start, finish, cnt)

        plsc.subcore_barrier()
        pltpu.sync_copy(acc.at[pl.ds(s * zrows, zrows)],
                        out.at[c, pl.ds(s * zrows, zrows)])

    return agg_kernel


@functools.lru_cache(maxsize=None)
def _make_sc_dec(nb):
    ep = _NW * nb * _B
    nph = -(-nb // _RPH)

    @functools.partial(
        pl.kernel,
        out_type=jax.ShapeDtypeStruct((ep,), jnp.float32),
        mesh=_mesh(),
        scratch_types=[
            pltpu.VMEM((_RPH, _B), jnp.int32),
            pltpu.VMEM((_RPH, _B), jnp.int32),
            pltpu.VMEM((_B, _D), jnp.float32),
            pltpu.VMEM((_B, _D), jnp.float32),
            pltpu.VMEM((_B, _D), jnp.float32),
            pltpu.VMEM((_B, _D), jnp.float32),
            pltpu.VMEM((_B,), jnp.float32),
            pltpu.VMEM((_D,), jnp.float32),
            pltpu.VMEM((_L,), jnp.float32),
            pltpu.SemaphoreType.DMA,
            pltpu.SemaphoreType.DMA,
        ],
    )
    def dec_kernel(pp, qq, sidx3, didx3, wd2, bias, out,
                   sidx_v, didx_v, rs0_v, rd0_v, rs1_v, rd1_v,
                   ov, wv, bv, sem_s, sem_d):
        c = lax.axis_index("c")
        s = lax.axis_index("s")
        wid = c * _NS + s
        base = wid * nb * _B
        ng = _B // _L
        nch = _D // _L
        pltpu.sync_copy(wd2, wv)
        pltpu.sync_copy(bias, bv)
        b0 = bv[...]
        wch = [wv[pl.ds(cc * _L, _L)] for cc in range(nch)]
        lane = lax.iota(jnp.int32, _L)
        perms = [(lane + sh) & (_L - 1) for sh in (8, 4, 2, 1)]

        rsb = (rs0_v, rs1_v)
        rdb = (rd0_v, rd1_v)
        for ph in range(nph):
            pb = ph * _RPH
            cnt = min(_RPH, nb - pb)
            pltpu.sync_copy(sidx3.at[wid, pl.ds(pb, cnt)],
                            sidx_v.at[pl.ds(0, cnt)])
            pltpu.sync_copy(didx3.at[wid, pl.ds(pb, cnt)],
                            didx_v.at[pl.ds(0, cnt)])

            def start(j, buf):
                pltpu.async_copy(pp.at[sidx_v.at[j]], rsb[buf], sem_s)
                pltpu.async_copy(qq.at[didx_v.at[j]], rdb[buf], sem_d)

            def finish(j, buf):
                pltpu.make_async_copy(pp.at[sidx_v.at[j]], rsb[buf],
                                      sem_s).wait()
                pltpu.make_async_copy(qq.at[didx_v.at[j]], rdb[buf],
                                      sem_d).wait()
                rs = rsb[buf]
                rd = rdb[buf]

                def gbody(g, carry):
                    res = b0
                    for jj in range(_L):
                        r = g * _L + jj
                        parts = [None] * 4
                        for cc in range(nch):
                            sl = pl.ds(cc * _L, _L)
                            t = (jnp.maximum(rs[r, sl] + rd[r, sl], 0.0)
                                 * wch[cc])
                            k4 = cc % 4
                            parts[k4] = (t if parts[k4] is None
                                         else parts[k4] + t)
                        acc = (parts[0] + parts[1]) + (parts[2] + parts[3])
                        for pidx in perms:
                            acc = acc + acc[pidx]
                        res = jnp.where(lane == jj, res + acc, res)
                    ov[pl.ds(g * _L, _L)] = res
                    return carry

                lax.fori_loop(0, ng, gbody, 0)
                pltpu.sync_copy(
                    ov, out.at[pl.ds(base + (pb + j) * _B, _B)])

            start(0, 0)
            _ring(start, finish, cnt)

    return dec_kernel


def _dcol(dt_ref):
    return lax.rsqrt(1.0 + dt_ref[:, 0:1] + dt_ref[:, 1:2])


def _k1_body(x_ref, w_ref, dt_ref, o_ref):
    o_ref[...] = _dcol(dt_ref) * jnp.dot(
        x_ref[...], w_ref[...], preferred_element_type=jnp.float32)


def _k2_body(p0_ref, p1_ref, y1_ref, dt_ref, w_ref, b_ref, o_ref):
    d = _dcol(dt_ref)
    h = jnp.maximum(
        d * (p0_ref[0] + p1_ref[0] + y1_ref[...]) + b_ref[...], 0.0)
    o_ref[...] = d * jnp.dot(h, w_ref[...],
                             preferred_element_type=jnp.float32)


def _k3_body(q0_ref, q1_ref, y2_ref, dt_ref, b_ref, wa_ref, wb_ref, bd1_ref,
             p_ref, qo_ref):
    emb = (_dcol(dt_ref)
           * (q0_ref[0] + q1_ref[0] + y2_ref[...]) + b_ref[...])
    p_ref[...] = jnp.dot(emb, wa_ref[...], precision=lax.Precision.HIGHEST,
                         preferred_element_type=jnp.float32) + bd1_ref[...]
    qo_ref[...] = jnp.dot(emb, wb_ref[...], precision=lax.Precision.HIGHEST,
                          preferred_element_type=jnp.float32)


def _row_spec(bn, width):
    return pl.BlockSpec((bn, width), lambda m: (m, 0))


def _full_spec(shape):
    return pl.BlockSpec(shape, lambda m: tuple(0 for _ in shape))


def _part_spec(core):
    return pl.BlockSpec((1, _BN, _D), lambda m: (core, m, 0))


def _tc_k1(x, w1, dt):
    return pl.pallas_call(
        _k1_body,
        grid=(_N // _BN,),
        in_specs=[_row_spec(_BN, _D), _full_spec((_D, _D)),
                  _row_spec(_BN, 2)],
        out_specs=_row_spec(_BN, _D),
        out_shape=jax.ShapeDtypeStruct((_N, _D), jnp.float32),
    )(x, w1, dt)


def _tc_k2(p, y1, dt, w2, b1):
    return pl.pallas_call(
        _k2_body,
        grid=(_N // _BN,),
        in_specs=[_part_spec(0), _part_spec(1), _row_spec(_BN, _D),
                  _row_spec(_BN, 2), _full_spec((_D, _D)),
                  _full_spec((1, _D))],
        out_specs=_row_spec(_BN, _D),
        out_shape=jax.ShapeDtypeStruct((_N, _D), jnp.float32),
    )(p, p, y1, dt, w2, b1)


def _tc_k3(q, y2, dt, b2, wa, wb, bd1):
    return pl.pallas_call(
        _k3_body,
        grid=(_N // _BN,),
        in_specs=[_part_spec(0), _part_spec(1), _row_spec(_BN, _D),
                  _row_spec(_BN, 2), _full_spec((1, _D)),
                  _full_spec((_D, _D)), _full_spec((_D, _D)),
                  _full_spec((1, _D))],
        out_specs=(_row_spec(_BN, _D), _row_spec(_BN, _D)),
        out_shape=(jax.ShapeDtypeStruct((_N, _D), jnp.float32),
                   jax.ShapeDtypeStruct((_N, _D), jnp.float32)),
    )(q, q, y2, dt, b2, wa, wb, bd1)


def _pad_split(idx2, n_batches, dst_dummy, bsize=_B):
    e = idx2.shape[1]
    ep = _NW * n_batches * bsize
    ar = jnp.arange(ep - e, dtype=jnp.int32)
    pad_a = (ar * 7919) % _N
    if dst_dummy:
        pad_b = _N + (ar % _NS)
    else:
        pad_b = (ar * 104729) % _N
    a = jnp.concatenate([idx2[0], pad_a]).reshape(_NW, n_batches, bsize)
    b = jnp.concatenate([idx2[1], pad_b]).reshape(_NW, n_batches, bsize)
    return a, b


def kernel(x, edge_index, edge_label_index, W1, b1, W2, b2,
           Wd1, bd1, Wd2, bd2):
    e = edge_index.shape[1]
    ep = edge_label_index.shape[1]
    nb = -(-e // (_NW * _B))
    nbp = -(-ep // (_NW * _B))

    src3, dst3 = _pad_split(edge_index, nb, dst_dummy=True)
    sl3, dl3 = _pad_split(edge_label_index, nbp, dst_dummy=False)

    degp = _make_sc_deg(nb)(dst3).reshape(_NC, _DEG_ACC)
    dt = jnp.transpose(degp)[:_N]

    y1 = _tc_k1(x, W1, dt)
    p = _make_sc_agg(nb)(y1, src3, dst3)
    y2 = _tc_k2(p, y1, dt, W2, b1.reshape(1, _D))
    q = _make_sc_agg(nb)(y2, src3, dst3)
    pmat, qmat = _tc_k3(q, y2, dt, b2.reshape(1, _D),
                        Wd1[:_D], Wd1[_D:], bd1.reshape(1, _D))

    bias = jnp.broadcast_to(bd2, (_L,)).astype(jnp.float32)
    predp = _make_sc_dec(nbp)(pmat, qmat, sl3, dl3, Wd2[:, 0], bias)
    return predp[:ep]


# speedup vs baseline: 1.0083x; 1.0083x over previous
import functools

import jax
import jax.numpy as jnp
from jax import lax
from jax.experimental import pallas as pl
from jax.experimental.pallas import tpu as pltpu
from jax.experimental.pallas import tpu_sc as plsc

_N = 10000
_D = 128

_NC = 2
_NS = 16
_NW = _NC * _NS
_L = 16

_B = 128
_ROWS_ACC = 10112
_DEG_ACC = 10240
_RPH = 48
_BN = 2000


def _mesh():
    return plsc.VectorSubcoreMesh(
        core_axis_name="c", subcore_axis_name="s",
        num_cores=_NC, num_subcores=_NS)


def _ring(start, finish, cnt):
    m1 = (cnt - 1) // 2

    def body(i, carry):
        j = 2 * i
        start(j + 1, 1)
        finish(j, 0)
        start(j + 2, 0)
        finish(j + 1, 1)
        return carry

    if m1 > 0:
        lax.fori_loop(0, m1, body, 0)
    if cnt % 2 == 1:
        finish(cnt - 1, 0)
    else:
        if cnt >= 2:
            start(cnt - 1, 1)
            finish(cnt - 2, 0)
            finish(cnt - 1, 1)
        else:
            finish(0, 0)


@functools.lru_cache(maxsize=None)
def _make_sc_deg(nb):
    zlen = _DEG_ACC // _NS

    @functools.partial(
        pl.kernel,
        out_type=jax.ShapeDtypeStruct((_NC * _DEG_ACC,), jnp.float32),
        mesh=_mesh(),
        scratch_types=[
            pltpu.VMEM_SHARED((_DEG_ACC,), jnp.float32),
            pltpu.VMEM((nb, _B), jnp.int32),
            pltpu.VMEM((_B,), jnp.float32),
            pltpu.VMEM((zlen,), jnp.float32),
        ],
    )
    def deg_kernel(dst3, out, acc, idx_v, ones_v, zbuf_v):
        c = lax.axis_index("c")
        s = lax.axis_index("s")
        wid = c * _NS + s
        for k in range(_B // _L):
            ones_v[pl.ds(k * _L, _L)] = jnp.ones((_L,), jnp.float32)

        def zb(i, carry):
            zbuf_v[pl.ds(i * _L, _L)] = jnp.zeros((_L,), jnp.float32)
            return carry

        lax.fori_loop(0, zlen // _L, zb, 0)
        pltpu.sync_copy(zbuf_v, acc.at[pl.ds(s * zlen, zlen)])
        plsc.subcore_barrier()
        pltpu.sync_copy(dst3.at[wid], idx_v)

        def body(i, carry):
            pltpu.sync_copy(ones_v, acc.at[idx_v.at[i]], add=True)
            return carry

        lax.fori_loop(0, nb, body, 0)
        plsc.subcore_barrier()
        pltpu.sync_copy(acc.at[pl.ds(s * zlen, zlen)],
                        out.at[pl.ds(c * _DEG_ACC + s * zlen, zlen)])

    return deg_kernel


@functools.lru_cache(maxsize=None)
def _make_sc_agg(nb):
    zrows = _ROWS_ACC // _NS
    nph = -(-nb // _RPH)

    @functools.partial(
        pl.kernel,
        out_type=jax.ShapeDtypeStruct((_NC, _ROWS_ACC, _D), jnp.float32),
        mesh=_mesh(),
        scratch_types=[
            pltpu.VMEM_SHARED((_ROWS_ACC, _D), jnp.float32),
            pltpu.VMEM((_RPH, _B), jnp.int32),
            pltpu.VMEM((_RPH, _B), jnp.int32),
            pltpu.VMEM((_B, _D), jnp.float32),
            pltpu.VMEM((_B, _D), jnp.float32),
            pltpu.SemaphoreType.DMA,
        ],
    )
    def agg_kernel(y, src3, dst3, out, acc, sidx_v, didx_v, r0_v, r1_v, sem):
        c = lax.axis_index("c")
        s = lax.axis_index("s")
        wid = c * _NS + s

        def zb(i, carry):
            for k in range(_D // _L):
                r0_v[i, pl.ds(k * _L, _L)] = jnp.zeros((_L,), jnp.float32)
            return carry

        lax.fori_loop(0, _B, zb, 0)
        for j in range(zrows // _B):
            pltpu.sync_copy(r0_v, acc.at[pl.ds(s * zrows + j * _B, _B)])
        pltpu.sync_copy(r0_v.at[pl.ds(0, zrows % _B)],
                        acc.at[pl.ds(s * zrows + (zrows // _B) * _B,
                                     zrows % _B)])
        plsc.subcore_barrier()

        bufs = (r0_v, r1_v)
        for ph in range(nph):
            pb = ph * _RPH
            cnt = min(_RPH, nb - pb)
            pltpu.sync_copy(src3.at[wid, pl.ds(pb, cnt)],
                            sidx_v.at[pl.ds(0, cnt)])
            pltpu.sync_copy(dst3.at[wid, pl.ds(pb, cnt)],
                            didx_v.at[pl.ds(0, cnt)])

            def start(j, buf):
                pltpu.async_copy(y.at[sidx_v.at[j]], bufs[buf], sem)

            def finish(j, buf):
                pltpu.make_async_copy(y.at[sidx_v.at[j]], bufs[buf],
                                      sem).wait()
                pltpu.sync_copy(bufs[buf], acc.at[didx_v.at[j]], add=True)

            start(0, 0)
            _ring(start, finish, cnt)

        plsc.subcore_barrier()
        pltpu.sync_copy(acc.at[pl.ds(s * zrows, zrows)],
                        out.at[c, pl.ds(s * zrows, zrows)])

    return agg_kernel


@functools.lru_cache(maxsize=None)
def _make_sc_dec(nb):
    ep = _NW * nb * _B
    nph = -(-nb // _RPH)

    @functools.partial(
        pl.kernel,
        out_type=jax.ShapeDtypeStruct((ep,), jnp.float32),
        mesh=_mesh(),
        scratch_types=[
            pltpu.VMEM((_RPH, _B), jnp.int32),
            pltpu.VMEM((_RPH, _B), jnp.int32),
            pltpu.VMEM((_B, _D), jnp.float32),
            pltpu.VMEM((_B, _D), jnp.float32),
            pltpu.VMEM((_B, _D), jnp.float32),
            pltpu.VMEM((_B, _D), jnp.float32),
            pltpu.VMEM((_B,), jnp.float32),
            pltpu.VMEM((_D,), jnp.float32),
            pltpu.VMEM((_L,), jnp.float32),
            pltpu.SemaphoreType.DMA,
            pltpu.SemaphoreType.DMA,
        ],
    )
    def dec_kernel(pp, qq, sidx3, didx3, wd2, bias, out,
                   sidx_v, didx_v, rs0_v, rd0_v, rs1_v, rd1_v,
                   ov, wv, bv, sem_s, sem_d):
        c = lax.axis_index("c")
        s = lax.axis_index("s")
        wid = c * _NS + s
        base = wid * nb * _B
        ng = _B // _L
        nch = _D // _L
        pltpu.sync_copy(wd2, wv)
        pltpu.sync_copy(bias, bv)
        b0 = bv[...]
        wch = [wv[pl.ds(cc * _L, _L)] for cc in range(nch)]
        lane = lax.iota(jnp.int32, _L)
        perms = [(lane + sh) & (_L - 1) for sh in (8, 4, 2, 1)]

        rsb = (rs0_v, rs1_v)
        rdb = (rd0_v, rd1_v)
        for ph in range(nph):
            pb = ph * _RPH
            cnt = min(_RPH, nb - pb)
            pltpu.sync_copy(sidx3.at[wid, pl.ds(pb, cnt)],
                            sidx_v.at[pl.ds(0, cnt)])
            pltpu.sync_copy(didx3.at[wid, pl.ds(pb, cnt)],
                            didx_v.at[pl.ds(0, cnt)])

            def start(j, buf):
                pltpu.async_copy(pp.at[sidx_v.at[j]], rsb[buf], sem_s)
                pltpu.async_copy(qq.at[didx_v.at[j]], rdb[buf], sem_d)

            def finish(j, buf):
                pltpu.make_async_copy(pp.at[sidx_v.at[j]], rsb[buf],
                                      sem_s).wait()
                pltpu.make_async_copy(qq.at[didx_v.at[j]], rdb[buf],
                                      sem_d).wait()
                rs = rsb[buf]
                rd = rdb[buf]

                def gbody(g, carry):
                    res = b0
                    for jj in range(_L):
                        r = g * _L + jj
                        parts = [None] * 4
                        for cc in range(nch):
                            sl = pl.ds(cc * _L, _L)
                            t = (jnp.maximum(rs[r, sl] + rd[r, sl], 0.0)
                                 * wch[cc])
                            k4 = cc % 4
                            parts[k4] = (t if parts[k4] is None
                                         else parts[k4] + t)
                        acc = (parts[0] + parts[1]) + (parts[2] + parts[3])
                        for pidx in perms:
                            acc = acc + acc[pidx]
                        res = jnp.where(lane == jj, res + acc, res)
                    ov[pl.ds(g * _L, _L)] = res
                    return carry

                lax.fori_loop(0, ng, gbody, 0)
                pltpu.sync_copy(
                    ov, out.at[pl.ds(base + (pb + j) * _B, _B)])

            start(0, 0)
            _ring(start, finish, cnt)

    return dec_kernel


def _dcol(dt_ref):
    return lax.rsqrt(1.0 + dt_ref[:, 0:1] + dt_ref[:, 1:2])


def _k1_body(x_ref, w_ref, dt_ref, o_ref):
    o_ref[...] = _dcol(dt_ref) * jnp.dot(
        x_ref[...], w_ref[...], preferred_element_type=jnp.float32)


def _k2_body(p0_ref, p1_ref, y1_ref, dt_ref, w_ref, b_ref, o_ref):
    d = _dcol(dt_ref)
    h = jnp.maximum(
        d * (p0_ref[0] + p1_ref[0] + y1_ref[...]) + b_ref[...], 0.0)
    o_ref[...] = d * jnp.dot(h, w_ref[...],
                             preferred_element_type=jnp.float32)


def _k3_body(q0_ref, q1_ref, y2_ref, dt_ref, b_ref, wa_ref, wb_ref, bd1_ref,
             p_ref, qo_ref):
    emb = (_dcol(dt_ref)
           * (q0_ref[0] + q1_ref[0] + y2_ref[...]) + b_ref[...])
    p_ref[...] = jnp.dot(emb, wa_ref[...],
                         preferred_element_type=jnp.float32) + bd1_ref[...]
    qo_ref[...] = jnp.dot(emb, wb_ref[...],
                          preferred_element_type=jnp.float32)


def _row_spec(bn, width):
    return pl.BlockSpec((bn, width), lambda m: (m, 0))


def _full_spec(shape):
    return pl.BlockSpec(shape, lambda m: tuple(0 for _ in shape))


def _part_spec(core):
    return pl.BlockSpec((1, _BN, _D), lambda m: (core, m, 0))


def _tc_k1(x, w1, dt):
    return pl.pallas_call(
        _k1_body,
        grid=(_N // _BN,),
        in_specs=[_row_spec(_BN, _D), _full_spec((_D, _D)),
                  _row_spec(_BN, 2)],
        out_specs=_row_spec(_BN, _D),
        out_shape=jax.ShapeDtypeStruct((_N, _D), jnp.float32),
    )(x, w1, dt)


def _tc_k2(p, y1, dt, w2, b1):
    return pl.pallas_call(
        _k2_body,
        grid=(_N // _BN,),
        in_specs=[_part_spec(0), _part_spec(1), _row_spec(_BN, _D),
                  _row_spec(_BN, 2), _full_spec((_D, _D)),
                  _full_spec((1, _D))],
        out_specs=_row_spec(_BN, _D),
        out_shape=jax.ShapeDtypeStruct((_N, _D), jnp.float32),
    )(p, p, y1, dt, w2, b1)


def _tc_k3(q, y2, dt, b2, wa, wb, bd1):
    return pl.pallas_call(
        _k3_body,
        grid=(_N // _BN,),
        in_specs=[_part_spec(0), _part_spec(1), _row_spec(_BN, _D),
                  _row_spec(_BN, 2), _full_spec((1, _D)),
                  _full_spec((_D, _D)), _full_spec((_D, _D)),
                  _full_spec((1, _D))],
        out_specs=(_row_spec(_BN, _D), _row_spec(_BN, _D)),
        out_shape=(jax.ShapeDtypeStruct((_N, _D), jnp.float32),
                   jax.ShapeDtypeStruct((_N, _D), jnp.float32)),
    )(q, q, y2, dt, b2, wa, wb, bd1)


def _pad_split(idx2, n_batches, dst_dummy, bsize=_B):
    e = idx2.shape[1]
    ep = _NW * n_batches * bsize
    ar = jnp.arange(ep - e, dtype=jnp.int32)
    pad_a = (ar * 7919) % _N
    if dst_dummy:
        pad_b = _N + (ar % _NS)
    else:
        pad_b = (ar * 104729) % _N
    a = jnp.concatenate([idx2[0], pad_a]).reshape(_NW, n_batches, bsize)
    b = jnp.concatenate([idx2[1], pad_b]).reshape(_NW, n_batches, bsize)
    return a, b


def kernel(x, edge_index, edge_label_index, W1, b1, W2, b2,
           Wd1, bd1, Wd2, bd2):
    e = edge_index.shape[1]
    ep = edge_label_index.shape[1]
    nb = -(-e // (_NW * _B))
    nbp = -(-ep // (_NW * _B))

    src3, dst3 = _pad_split(edge_index, nb, dst_dummy=True)
    sl3, dl3 = _pad_split(edge_label_index, nbp, dst_dummy=False)

    degp = _make_sc_deg(nb)(dst3).reshape(_NC, _DEG_ACC)
    dt = jnp.transpose(degp)[:_N]

    y1 = _tc_k1(x, W1, dt)
    p = _make_sc_agg(nb)(y1, src3, dst3)
    y2 = _tc_k2(p, y1, dt, W2, b1.reshape(1, _D))
    q = _make_sc_agg(nb)(y2, src3, dst3)
    pmat, qmat = _tc_k3(q, y2, dt, b2.reshape(1, _D),
                        Wd1[:_D], Wd1[_D:], bd1.reshape(1, _D))

    bias = jnp.broadcast_to(bd2, (_L,)).astype(jnp.float32)
    predp = _make_sc_dec(nbp)(pmat, qmat, sl3, dl3, Wd2[:, 0], bias)
    return predp[:ep]
